# Initial kernel scaffold; baseline (speedup 1.0000x reference)
import functools

import jax
import jax.numpy as jnp
from jax import lax
from jax.experimental import pallas as pl
from jax.experimental.pallas import tpu as pltpu
from jax.experimental.pallas import tpu_sc as plsc

N_NODES = 100000
D = 128
M = 600000

NC = 2
NS = 16
NW = NC * NS

CHUNK = 128
NBUF = 3
CHUNKS_PER_W = 147
GROUPS = CHUNKS_PER_W // NBUF
M_PAD = NW * CHUNKS_PER_W * CHUNK


def _gather_body(x_hbm, idx_hbm, out_hbm, idx_v, b0, b1, b2,
                 g0, g1, g2, w0, w1, w2):
    bufs = (b0, b1, b2)
    gsems = (g0, g1, g2)
    wsems = (w0, w1, w2)

    wid = lax.axis_index("s") * NC + lax.axis_index("c")
    base = wid * CHUNKS_PER_W

    pltpu.sync_copy(idx_hbm.at[pl.ds(base, CHUNKS_PER_W)], idx_v)

    def start_gather(j, b):
        pltpu.async_copy(x_hbm.at[idx_v.at[j]], bufs[b], gsems[b])

    def wait_gather(j, b):
        pltpu.make_async_copy(x_hbm.at[idx_v.at[j]], bufs[b], gsems[b]).wait()

    def start_write(j, b):
        pltpu.async_copy(bufs[b], out_hbm.at[pl.ds((base + j) * CHUNK, CHUNK)],
                         wsems[b])

    def wait_write(j, b):
        pltpu.make_async_copy(
            bufs[b], out_hbm.at[pl.ds((base + j) * CHUNK, CHUNK)],
            wsems[b]).wait()

    for b in range(NBUF):
        start_gather(b, b)

    def group(g, carry):
        for b in range(NBUF):
            j = g * NBUF + b
            wait_gather(j, b)
            start_write(j, b)
            wait_write(j, b)
            start_gather(j + NBUF, b)
        return carry

    lax.fori_loop(0, GROUPS - 1, group, 0)

    for b in range(NBUF):
        j = (GROUPS - 1) * NBUF + b
        wait_gather(j, b)
        start_write(j, b)
    for b in range(NBUF):
        j = (GROUPS - 1) * NBUF + b
        wait_write(j, b)


_gather_call = functools.partial(
    pl.kernel,
    out_type=jax.ShapeDtypeStruct((M_PAD, D), jnp.float32),
    mesh=plsc.VectorSubcoreMesh(core_axis_name="c", subcore_axis_name="s"),
    scratch_types=[
        pltpu.VMEM((CHUNKS_PER_W, CHUNK), jnp.int32),
        pltpu.VMEM((CHUNK, D), jnp.float32),
        pltpu.VMEM((CHUNK, D), jnp.float32),
        pltpu.VMEM((CHUNK, D), jnp.float32),
        pltpu.SemaphoreType.DMA,
        pltpu.SemaphoreType.DMA,
        pltpu.SemaphoreType.DMA,
        pltpu.SemaphoreType.DMA,
        pltpu.SemaphoreType.DMA,
        pltpu.SemaphoreType.DMA,
    ],
)(_gather_body)


def kernel(x, seed_inverse_ids):
    idx = seed_inverse_ids.astype(jnp.int32)
    idx = jnp.pad(idx, (0, M_PAD - M)).reshape(M_PAD // CHUNK, CHUNK)
    out = _gather_call(x, idx)
    return out[:M]


# speedup vs baseline: 4.5257x; 4.5257x over previous
import functools

import jax
import jax.numpy as jnp
from jax import lax
from jax.experimental import pallas as pl
from jax.experimental.pallas import tpu as pltpu
from jax.experimental.pallas import tpu_sc as plsc

N_NODES = 100000
D = 128
M = 600000

NC = 2
NS = 16
NW = NC * NS

CHUNK = 128
NBUF = 3
CHUNKS_PER_W = 147
GROUPS = CHUNKS_PER_W // NBUF
M_PAD = NW * CHUNKS_PER_W * CHUNK


def _gather_body(x_hbm, idx_hbm, out_hbm, idx_v, b0, b1, b2,
                 g0, g1, g2, w0, w1, w2):
    bufs = (b0, b1, b2)
    gsems = (g0, g1, g2)
    wsems = (w0, w1, w2)

    wid = lax.axis_index("s") * NC + lax.axis_index("c")
    base = wid * CHUNKS_PER_W

    pltpu.sync_copy(idx_hbm.at[wid], idx_v)

    def start_gather(j, b):
        pltpu.async_copy(x_hbm.at[idx_v.at[j]], bufs[b], gsems[b])

    def wait_gather(j, b):
        pltpu.make_async_copy(x_hbm.at[idx_v.at[j]], bufs[b], gsems[b]).wait()

    def start_write(j, b):
        pltpu.async_copy(bufs[b], out_hbm.at[pl.ds((base + j) * CHUNK, CHUNK)],
                         wsems[b])

    def wait_write(j, b):
        pltpu.make_async_copy(
            bufs[b], out_hbm.at[pl.ds((base + j) * CHUNK, CHUNK)],
            wsems[b]).wait()

    for b in range(NBUF):
        start_gather(b, b)

    def group(g, carry):
        for b in range(NBUF):
            j = g * NBUF + b
            wait_gather(j, b)
            start_write(j, b)
            wait_write(j, b)
            start_gather(j + NBUF, b)
        return carry

    lax.fori_loop(0, GROUPS - 1, group, 0)

    for b in range(NBUF):
        j = (GROUPS - 1) * NBUF + b
        wait_gather(j, b)
        start_write(j, b)
    for b in range(NBUF):
        j = (GROUPS - 1) * NBUF + b
        wait_write(j, b)


_gather_call = functools.partial(
    pl.kernel,
    out_type=jax.ShapeDtypeStruct((M_PAD, D), jnp.float32),
    mesh=plsc.VectorSubcoreMesh(core_axis_name="c", subcore_axis_name="s"),
    scratch_types=[
        pltpu.VMEM((CHUNKS_PER_W, CHUNK), jnp.int32),
        pltpu.VMEM((CHUNK, D), jnp.float32),
        pltpu.VMEM((CHUNK, D), jnp.float32),
        pltpu.VMEM((CHUNK, D), jnp.float32),
        pltpu.SemaphoreType.DMA,
        pltpu.SemaphoreType.DMA,
        pltpu.SemaphoreType.DMA,
        pltpu.SemaphoreType.DMA,
        pltpu.SemaphoreType.DMA,
        pltpu.SemaphoreType.DMA,
    ],
)(_gather_body)


def kernel(x, seed_inverse_ids):
    idx = seed_inverse_ids.astype(jnp.int32)
    idx = jnp.pad(idx, (0, M_PAD - M)).reshape(NW, CHUNKS_PER_W, CHUNK)
    out = _gather_call(x, idx)
    return out[:M]
